# Initial kernel scaffold; baseline (speedup 1.0000x reference)
import jax
import jax.numpy as jnp
from jax.experimental import pallas as pl


def kernel(x, x_pe, edge_index, edge_attr, edge_pe, mod_w1, mod_b1, mod_w2, mod_b2, lin_w, lin_b, theta1, theta2, ln1_g, ln1_b, ffn_w1, ffn_b1, ffn_w2, ffn_b2, ln2_g, ln2_b):
    raise NotImplementedError("write your pallas kernel here")


# speedup vs baseline: 2.3410x; 2.3410x over previous
import functools

import jax
import jax.numpy as jnp
from jax import lax
from jax.experimental import pallas as pl
from jax.experimental.pallas import tpu as pltpu
from jax.experimental.pallas import tpu_sc as plsc

N = 10000
E = 640000
NF = 144
EF = 32
MOD_H = 16
OUT = 128
FFN_H = 512

N_PAD = 10240
NW = 32
CHUNK = 64
NCHUNK = 314
EPT = CHUNK * NCHUNK
E_PAD = NW * EPT
EBLK = 1024
ROWS_PER_TILE = N_PAD // 16
NBLK = 1000


def _gelu(v):
    return 0.5 * v * (1.0 + lax.erf(v * 0.7071067811865476))


def _ew_body(ec_ref, w1_ref, b1_ref, w2_ref, b2_ref, out_ref):
    ec = ec_ref[...]
    h = jnp.dot(ec, w1_ref[...], preferred_element_type=jnp.float32) + b1_ref[...]
    h = _gelu(h)
    out_ref[...] = (
        jnp.dot(h, w2_ref[...], preferred_element_type=jnp.float32) + b2_ref[...]
    )


def _edge_mlp(ec_pad, w1, b1, w2, b2):
    grid = E_PAD // EBLK
    return pl.pallas_call(
        _ew_body,
        grid=(grid,),
        in_specs=[
            pl.BlockSpec((EBLK, EF), lambda i: (i, 0)),
            pl.BlockSpec((EF, MOD_H), lambda i: (0, 0)),
            pl.BlockSpec((1, MOD_H), lambda i: (0, 0)),
            pl.BlockSpec((MOD_H, NF), lambda i: (0, 0)),
            pl.BlockSpec((1, NF), lambda i: (0, 0)),
        ],
        out_specs=pl.BlockSpec((EBLK, NF), lambda i: (i, 0)),
        out_shape=jax.ShapeDtypeStruct((E_PAD, NF), jnp.float32),
    )(ec_pad, w1, b1, w2, b2)


def _sc_body(
    xc_hbm, src_hbm, dst_hbm, ew_hbm, z144_hbm, z16_hbm,
    sums_hbm, cd_hbm,
    srcv, dstv, ewv, xrv, cntv, degv, acc, acc_cd, sem_ew, sem_g,
):
    c = lax.axis_index("c")
    s = lax.axis_index("s")
    wid = s * 2 + c

    row0 = s * ROWS_PER_TILE
    for j in range(ROWS_PER_TILE // CHUNK):
        r = row0 + j * CHUNK
        pltpu.sync_copy(z144_hbm, acc.at[pl.ds(r, CHUNK)])
        pltpu.sync_copy(z16_hbm, acc_cd.at[pl.ds(r, CHUNK)])

    lanes = lax.iota(jnp.int32, 16)
    onehot0 = jnp.where(lanes == 0, 1.0, 0.0).astype(jnp.float32)
    onehot1 = jnp.where(lanes == 1, 1.0, 0.0).astype(jnp.float32)

    def initrow(r, carry):
        cntv[r, pl.ds(0, 16)] = onehot0
        degv[r, pl.ds(0, 16)] = onehot1
        return carry

    lax.fori_loop(0, CHUNK, initrow, 0)
    plsc.subcore_barrier()

    base_w = wid * EPT

    def chunk_body(i, carry):
        base = base_w + i * CHUNK
        pltpu.sync_copy(src_hbm.at[pl.ds(base, CHUNK)], srcv)
        pltpu.sync_copy(dst_hbm.at[pl.ds(base, CHUNK)], dstv)
        cp_ew = pltpu.async_copy(ew_hbm.at[pl.ds(base, CHUNK)], ewv, sem_ew)
        cp_g = pltpu.async_copy(xc_hbm.at[srcv], xrv, sem_g)
        cp_ew.wait()
        cp_g.wait()

        def mrow(r, cc):
            for k in range(NF // 16):
                sl = pl.ds(k * 16, 16)
                xrv[r, sl] = xrv[r, sl] * ewv[r, sl]
            return cc

        lax.fori_loop(0, CHUNK, mrow, 0)
        pltpu.sync_copy(xrv, acc.at[dstv], add=True)
        pltpu.sync_copy(cntv, acc_cd.at[dstv], add=True)
        pltpu.sync_copy(degv, acc_cd.at[srcv], add=True)
        return carry

    lax.fori_loop(0, NCHUNK, chunk_body, 0)
    plsc.subcore_barrier()

    out0 = c * N_PAD + row0
    for j in range(ROWS_PER_TILE // CHUNK):
        r = row0 + j * CHUNK
        o = out0 + j * CHUNK
        pltpu.sync_copy(acc.at[pl.ds(r, CHUNK)], sums_hbm.at[pl.ds(o, CHUNK)])
        pltpu.sync_copy(acc_cd.at[pl.ds(r, CHUNK)], cd_hbm.at[pl.ds(o, CHUNK)])


def _sc_aggregate(xc_pad, src_pad, dst_pad, ew, z144, z16):
    mesh = plsc.VectorSubcoreMesh(core_axis_name="c", subcore_axis_name="s")
    fn = functools.partial(
        pl.kernel,
        mesh=mesh,
        compiler_params=pltpu.CompilerParams(use_tc_tiling_on_sc=False),
        out_type=[
            jax.ShapeDtypeStruct((2 * N_PAD, NF), jnp.float32),
            jax.ShapeDtypeStruct((2 * N_PAD, 16), jnp.float32),
        ],
        scratch_types=[
            pltpu.VMEM((CHUNK,), jnp.int32),
            pltpu.VMEM((CHUNK,), jnp.int32),
            pltpu.VMEM((CHUNK, NF), jnp.float32),
            pltpu.VMEM((CHUNK, NF), jnp.float32),
            pltpu.VMEM((CHUNK, 16), jnp.float32),
            pltpu.VMEM((CHUNK, 16), jnp.float32),
            pltpu.VMEM_SHARED((N_PAD, NF), jnp.float32),
            pltpu.VMEM_SHARED((N_PAD, 16), jnp.float32),
            pltpu.SemaphoreType.DMA,
            pltpu.SemaphoreType.DMA,
        ],
    )(_sc_body)
    return fn(xc_pad, src_pad, dst_pad, ew, z144, z16)


def _ln(v, g, b):
    mu = jnp.mean(v, axis=-1, keepdims=True)
    var = jnp.mean((v - mu) ** 2, axis=-1, keepdims=True)
    return (v - mu) * lax.rsqrt(var + 1e-5) * g + b


def _post_body(
    sums_ref, cd_ref, x_ref,
    lin_w_ref, lin_b_ref, th1_ref, th2_ref, ln1g_ref, ln1b_ref,
    fw1_ref, fb1_ref, fw2_ref, fb2_ref, ln2g_ref, ln2b_ref,
    out_ref,
):
    sums = sums_ref[0] + sums_ref[1]
    cd = cd_ref[0] + cd_ref[1]
    cnt = cd[:, 0:1]
    deg = cd[:, 1:2]
    aggr = sums / jnp.maximum(cnt, 1.0)
    out = jnp.dot(aggr, lin_w_ref[...], preferred_element_type=jnp.float32)
    out = out + lin_b_ref[...]
    deg_sqrt = jnp.sqrt(jnp.maximum(deg, 1.0))
    out = out * th1_ref[...] + deg_sqrt * (out * th2_ref[...])
    y = _ln(out, ln1g_ref[...], ln1b_ref[...])
    y = y + x_ref[...]
    h = jnp.dot(y, fw1_ref[...], preferred_element_type=jnp.float32) + fb1_ref[...]
    h = _gelu(h)
    z = jnp.dot(h, fw2_ref[...], preferred_element_type=jnp.float32) + fb2_ref[...]
    z = z + y
    out_ref[...] = _ln(z, ln2g_ref[...], ln2b_ref[...])


def _post(sums3, cd3, x, lin_w, lin_b, th1, th2, ln1g, ln1b,
          fw1, fb1, fw2, fb2, ln2g, ln2b):
    grid = N // NBLK
    full = lambda shape: pl.BlockSpec(shape, lambda i: tuple(0 for _ in shape))
    return pl.pallas_call(
        _post_body,
        grid=(grid,),
        in_specs=[
            pl.BlockSpec((2, NBLK, NF), lambda i: (0, i, 0)),
            pl.BlockSpec((2, NBLK, 16), lambda i: (0, i, 0)),
            pl.BlockSpec((NBLK, OUT), lambda i: (i, 0)),
            full((NF, OUT)),
            full((1, OUT)),
            full((1, OUT)),
            full((1, OUT)),
            full((1, OUT)),
            full((1, OUT)),
            full((OUT, FFN_H)),
            full((1, FFN_H)),
            full((FFN_H, OUT)),
            full((1, OUT)),
            full((1, OUT)),
            full((1, OUT)),
        ],
        out_specs=pl.BlockSpec((NBLK, OUT), lambda i: (i, 0)),
        out_shape=jax.ShapeDtypeStruct((N, OUT), jnp.float32),
    )(sums3, cd3, x, lin_w, lin_b, th1, th2, ln1g, ln1b,
      fw1, fb1, fw2, fb2, ln2g, ln2b)


def kernel(x, x_pe, edge_index, edge_attr, edge_pe, mod_w1, mod_b1, mod_w2,
           mod_b2, lin_w, lin_b, theta1, theta2, ln1_g, ln1_b, ffn_w1, ffn_b1,
           ffn_w2, ffn_b2, ln2_g, ln2_b):
    f32 = jnp.float32
    xc = jnp.concatenate([x, x_pe], axis=1)
    xc_pad = jnp.concatenate([xc, jnp.zeros((N_PAD - N, NF), f32)], axis=0)
    ec = jnp.concatenate([edge_attr, edge_pe], axis=1)
    ec_pad = jnp.concatenate([ec, jnp.zeros((E_PAD - E, EF), f32)], axis=0)
    pad_idx = jnp.full((E_PAD - E,), N, jnp.int32)
    src_pad = jnp.concatenate([edge_index[0], pad_idx])
    dst_pad = jnp.concatenate([edge_index[1], pad_idx])

    ew = _edge_mlp(ec_pad, mod_w1, mod_b1.reshape(1, -1), mod_w2,
                   mod_b2.reshape(1, -1))

    z144 = jnp.zeros((CHUNK, NF), f32)
    z16 = jnp.zeros((CHUNK, 16), f32)
    sums_f, cd_f = _sc_aggregate(xc_pad, src_pad, dst_pad, ew, z144, z16)

    return _post(
        sums_f.reshape(2, N_PAD, NF),
        cd_f.reshape(2, N_PAD, 16),
        x, lin_w,
        lin_b.reshape(1, -1), theta1.reshape(1, -1), theta2.reshape(1, -1),
        ln1_g.reshape(1, -1), ln1_b.reshape(1, -1),
        ffn_w1, ffn_b1.reshape(1, -1), ffn_w2, ffn_b2.reshape(1, -1),
        ln2_g.reshape(1, -1), ln2_b.reshape(1, -1),
    )
